# Initial kernel scaffold; baseline (speedup 1.0000x reference)
import jax
import jax.numpy as jnp
from jax.experimental import pallas as pl


def kernel(feats, segment_ids, W, b):
    raise NotImplementedError("write your pallas kernel here")


# speedup vs baseline: 3.9070x; 3.9070x over previous
import jax
import jax.numpy as jnp
from jax.experimental import pallas as pl

N = 100000
D = 128
B = 1024
R = 1000
NB = N // R


def _body(seg_ref, feats_ref, W_ref, b_ref, out_ref):
    i = pl.program_id(0)

    @pl.when(i == 0)
    def _init():
        out_ref[...] = jnp.zeros_like(out_ref)

    f = feats_ref[...]
    y = jnp.dot(f, W_ref[...], preferred_element_type=jnp.float32) + b_ref[0, 0]
    w = 1.0 / (1.0 + jnp.exp(-y))
    weighted = f * w
    seg = seg_ref[0, 0, :]
    onehot = (
        seg[None, :] == jax.lax.broadcasted_iota(jnp.int32, (B, R), 0)
    ).astype(jnp.float32)
    out_ref[...] += jnp.dot(onehot, weighted, preferred_element_type=jnp.float32)


def kernel(feats, segment_ids, W, b):
    seg3 = segment_ids.astype(jnp.int32).reshape(NB, 1, R)
    b2 = b.reshape(1, 1).astype(jnp.float32)
    return pl.pallas_call(
        _body,
        grid=(NB,),
        in_specs=[
            pl.BlockSpec((1, 1, R), lambda i: (i, 0, 0)),
            pl.BlockSpec((R, D), lambda i: (i, 0)),
            pl.BlockSpec((D, 1), lambda i: (0, 0)),
            pl.BlockSpec((1, 1), lambda i: (0, 0)),
        ],
        out_specs=pl.BlockSpec((B, D), lambda i: (0, 0)),
        out_shape=jax.ShapeDtypeStruct((B, D), jnp.float32),
    )(seg3, feats, W, b2)
